# Initial kernel scaffold; baseline (speedup 1.0000x reference)
import jax
import jax.numpy as jnp
from jax.experimental import pallas as pl


def kernel(x, vectors):
    raise NotImplementedError("write your pallas kernel here")


# speedup vs baseline: 1.1033x; 1.1033x over previous
import functools

import jax
import jax.numpy as jnp
from jax import lax
from jax.experimental import pallas as pl
from jax.experimental.pallas import tpu as pltpu
from jax.experimental.pallas import tpu_sc as plsc

VOCAB = 1000000
EMBED_DIM = 32
BATCH = 16384
HIST = 50
TOTAL = BATCH * HIST

_NUM_WORKERS = 32
_PER_WORKER = TOTAL // _NUM_WORKERS
_CHUNK = 1600
_NCHUNKS = _PER_WORKER // _CHUNK

_mesh = plsc.VectorSubcoreMesh(core_axis_name="c", subcore_axis_name="s")


@functools.partial(
    pl.kernel,
    mesh=_mesh,
    out_type=jax.ShapeDtypeStruct((TOTAL, EMBED_DIM), jnp.float32),
    scratch_types=[
        pltpu.VMEM((_CHUNK,), jnp.int32),
        pltpu.VMEM((_CHUNK, EMBED_DIM), jnp.float32),
        pltpu.SemaphoreType.DMA,
    ],
    compiler_params=pltpu.CompilerParams(use_tc_tiling_on_sc=False),
)
def _gather_kernel(idx_hbm, table_hbm, out_hbm, idx_v, rows_v, sem):
    wid = lax.axis_index("s") * 2 + lax.axis_index("c")
    base0 = wid * _PER_WORKER

    def body(i, carry):
        base = base0 + i * _CHUNK
        pltpu.sync_copy(idx_hbm.at[pl.ds(base, _CHUNK)], idx_v)
        pltpu.async_copy(table_hbm.at[idx_v], rows_v, sem).wait()
        pltpu.sync_copy(rows_v, out_hbm.at[pl.ds(base, _CHUNK)])
        return carry

    lax.fori_loop(0, _NCHUNKS, body, 0)


def kernel(x, vectors):
    idx = x.reshape(-1).astype(jnp.int32)
    out = _gather_kernel(idx, vectors)
    return out.reshape(BATCH, HIST, EMBED_DIM)


# speedup vs baseline: 1.1105x; 1.0065x over previous
import functools

import jax
import jax.numpy as jnp
from jax import lax
from jax.experimental import pallas as pl
from jax.experimental.pallas import tpu as pltpu
from jax.experimental.pallas import tpu_sc as plsc

VOCAB = 1000000
EMBED_DIM = 32
BATCH = 16384
HIST = 50
TOTAL = BATCH * HIST

_NUM_WORKERS = 32
_PER_WORKER = TOTAL // _NUM_WORKERS
_CHUNK = 1280
_NCHUNKS = _PER_WORKER // _CHUNK
_NBUF = 3

_mesh = plsc.VectorSubcoreMesh(core_axis_name="c", subcore_axis_name="s")


@functools.partial(
    pl.kernel,
    mesh=_mesh,
    out_type=jax.ShapeDtypeStruct((TOTAL, EMBED_DIM), jnp.float32),
    scratch_types=[
        pltpu.VMEM((_NBUF, _CHUNK), jnp.int32),
        pltpu.VMEM((_NBUF, _CHUNK, EMBED_DIM), jnp.float32),
        pltpu.SemaphoreType.DMA((_NBUF,)),
        pltpu.SemaphoreType.DMA((_NBUF,)),
    ],
    compiler_params=pltpu.CompilerParams(use_tc_tiling_on_sc=False),
)
def _gather_kernel(idx_hbm, table_hbm, out_hbm, idx_v, rows_v, gsems, osems):
    wid = lax.axis_index("s") * 2 + lax.axis_index("c")
    base0 = wid * _PER_WORKER

    def start_chunk(j):
        b = j % _NBUF
        pltpu.sync_copy(idx_hbm.at[pl.ds(base0 + j * _CHUNK, _CHUNK)],
                        idx_v.at[b])
        pltpu.make_async_copy(table_hbm.at[idx_v.at[b]], rows_v.at[b],
                              gsems.at[b]).start()

    def out_copy(i):
        b = i % _NBUF
        return pltpu.make_async_copy(
            rows_v.at[b],
            out_hbm.at[pl.ds(base0 + i * _CHUNK, _CHUNK)],
            osems.at[b])

    for j in range(_NBUF - 1):
        start_chunk(j)

    for i in range(_NCHUNKS):
        b = i % _NBUF
        j = i + _NBUF - 1
        if j < _NCHUNKS:
            if j - _NBUF >= 0:
                out_copy(j - _NBUF).wait()
            start_chunk(j)
        pltpu.make_async_copy(table_hbm.at[idx_v.at[b]], rows_v.at[b],
                              gsems.at[b]).wait()
        out_copy(i).start()

    for i in range(max(0, _NCHUNKS - _NBUF), _NCHUNKS):
        out_copy(i).wait()


def kernel(x, vectors):
    idx = x.reshape(-1).astype(jnp.int32)
    out = _gather_kernel(idx, vectors)
    return out.reshape(BATCH, HIST, EMBED_DIM)


# speedup vs baseline: 1.9299x; 1.7379x over previous
import functools

import jax
import jax.numpy as jnp
from jax import lax
from jax.experimental import pallas as pl
from jax.experimental.pallas import tpu as pltpu
from jax.experimental.pallas import tpu_sc as plsc

VOCAB = 1000000
EMBED_DIM = 32
BATCH = 16384
HIST = 50
TOTAL = BATCH * HIST

_NUM_WORKERS = 32
_PER_WORKER = TOTAL // _NUM_WORKERS
_CHUNK = 1280
_NCHUNKS = _PER_WORKER // _CHUNK
_NBUF = 3

_mesh = plsc.VectorSubcoreMesh(core_axis_name="c", subcore_axis_name="s")


@functools.partial(
    pl.kernel,
    mesh=_mesh,
    out_type=jax.ShapeDtypeStruct((TOTAL, EMBED_DIM), jnp.float32),
    scratch_types=[
        pltpu.VMEM((_NBUF, _CHUNK), jnp.int32),
        pltpu.VMEM((_NBUF, _CHUNK, EMBED_DIM), jnp.float32),
        pltpu.SemaphoreType.DMA((_NBUF,)),
        pltpu.SemaphoreType.DMA((_NBUF,)),
    ],
    compiler_params=pltpu.CompilerParams(use_tc_tiling_on_sc=False),
)
def _gather_kernel(idx_hbm, table_hbm, out_hbm, idx_v, rows_v, gsems, osems):
    wid = lax.axis_index("s") * 2 + lax.axis_index("c")
    base0 = wid * _PER_WORKER

    def start_chunk(j):
        b = j % _NBUF
        pltpu.sync_copy(idx_hbm.at[pl.ds(base0 + j * _CHUNK, _CHUNK)],
                        idx_v.at[b])
        pltpu.make_async_copy(table_hbm.at[idx_v.at[b]], rows_v.at[b],
                              gsems.at[b]).start()

    def out_copy(i):
        b = i % _NBUF
        return pltpu.make_async_copy(
            rows_v.at[b],
            out_hbm.at[pl.ds(base0 + i * _CHUNK, _CHUNK)],
            osems.at[b])

    for j in range(_NBUF - 1):
        start_chunk(j)

    for i in range(_NCHUNKS):
        b = i % _NBUF
        j = i + _NBUF - 1
        if j < _NCHUNKS:
            if j - _NBUF >= 0:
                out_copy(j - _NBUF).wait()
            start_chunk(j)
        pltpu.make_async_copy(table_hbm.at[idx_v.at[b]], rows_v.at[b],
                              gsems.at[b]).wait()
        out_copy(i).start()

    for i in range(max(0, _NCHUNKS - _NBUF), _NCHUNKS):
        out_copy(i).wait()


def kernel(x, vectors):
    idx = x.T.reshape(-1).astype(jnp.int32)
    out = _gather_kernel(idx, vectors)
    return out.reshape(HIST, BATCH, EMBED_DIM).transpose(1, 0, 2)
